# Initial kernel scaffold; baseline (speedup 1.0000x reference)
import jax
import jax.numpy as jnp
from jax.experimental import pallas as pl


def kernel(nuclei, params):
    raise NotImplementedError("write your pallas kernel here")


# speedup vs baseline: 1.8752x; 1.8752x over previous
import functools

import jax
import jax.numpy as jnp
import numpy as np
from jax.experimental import pallas as pl

_N = 128
_RBF_DIM = 32
_RBF_CUT = 10.0
_POS_CUT = 5.0
_N_RAD = 6
_N_SPH = 7
_CHARGES = np.tile(np.array([1, 1, 6, 8], dtype=np.int64), 32).astype(np.int32)
_S = 8
_NB = _N // _S
_SP = _S * _N

_ONEHOT = np.zeros((_N, 9), dtype=np.float32)
_ONEHOT[np.arange(_N), _CHARGES] = 1.0
_E6 = np.kron(np.eye(_N_RAD, dtype=np.float32), np.ones((1, _N_SPH), np.float32))
_E7 = np.kron(np.ones((1, _N_RAD), np.float32), np.eye(_N_SPH, dtype=np.float32))
_R = np.kron(np.eye(_S, dtype=np.float32), np.ones((_N, 1), np.float32))
_MASK = np.ones((_NB * _SP, 1), dtype=np.float32)
for _b in range(_NB):
    for _s in range(_S):
        _MASK[_b * _SP + _s * _N + (_b * _S + _s), 0] = 0.0


def _dot(a, b):
    return jax.lax.dot_general(
        a, b, (((1,), (0,)), ((), ())), precision=jax.lax.Precision.HIGHEST)


def _silu(x):
    return x * jax.nn.sigmoid(x)


def _edge_layer(nuc, A, B, We, V, c, consts):
    R, mask, karr = consts
    B_t = jnp.concatenate([B] * _S, axis=0)
    nr_t = jnp.concatenate([nuc] * _S, axis=0)
    out_blocks = []
    for b in range(_NB):
        Ablk = A[b * _S:(b + 1) * _S, :]
        nblk = nuc[b * _S:(b + 1) * _S, :]
        a_pairs = _dot(R, Ablk)
        s_pairs = _dot(R, nblk)
        d = s_pairs - nr_t
        r = jnp.sqrt(jnp.sum(d * d, axis=1, keepdims=True))
        e = (np.float32(np.sqrt(2.0 / _RBF_CUT))
             * jnp.sin(karr * (np.float32(np.pi / _RBF_CUT)) * r)
             / (r + 1e-8))
        pre = a_pairs + B_t + _dot(e, We)
        m = _dot(_silu(pre), V) + c
        m = m * mask[b * _SP:(b + 1) * _SP, :]
        s = jax.lax.dot_general(R, m, (((0,), (0,)), ((), ())),
                                precision=jax.lax.Precision.HIGHEST)
        out_blocks.append(s)
    msg = jnp.concatenate(out_blocks, axis=0) * np.float32(1.0 / (_N - 1))
    return msg


def _fwd_kernel(nuc_ref, emb_ref, nemb_ref,
                onehot_ref, r_ref, mask_ref, e6_ref, e7_ref,
                w0s_ref, w0r_ref, w0e_ref, b0_ref, v0_ref, c0_ref,
                u0a_ref, u0b_ref, ub0_ref, u0c_ref, uc0_ref,
                w1s_ref, w1r_ref, w1e_ref, b1_ref, v1_ref, c1_ref,
                u1a_ref, u1b_ref, ub1_ref, u1c_ref, uc1_ref,
                n0_ref, nb0_ref, n1_ref,
                g0_ref, gb0_ref, g1_ref, gb1_ref,
                node_out_ref, glob_out_ref):
    nuc = nuc_ref[...]
    onehot = onehot_ref[...]
    karr = (jax.lax.broadcasted_iota(jnp.int32, (1, _RBF_DIM), 1)
            .astype(jnp.float32) + 1.0)
    consts = (r_ref[...], mask_ref[...], karr)

    center = nuc - jnp.mean(nuc, axis=0, keepdims=True)
    r2 = jnp.sum(center * center, axis=1, keepdims=True)
    r = jnp.sqrt(r2)
    inv = 1.0 / (r + 1e-8)
    u = center * inv
    nrad = (jax.lax.broadcasted_iota(jnp.int32, (1, _N_RAD), 1)
            .astype(jnp.float32) + 1.0)
    rad = (np.float32(np.sqrt(2.0 / _POS_CUT))
           * jnp.sin(nrad * np.float32(np.pi / _POS_CUT) * r) * inv)
    ux, uy, uz = u[:, 0:1], u[:, 1:2], u[:, 2:3]
    sph = jnp.concatenate(
        [jnp.ones_like(ux), ux, uy, uz, ux * uy, uy * uz, ux * uz], axis=1)
    pos = _dot(rad, e6_ref[...]) * _dot(sph, e7_ref[...])
    n0 = jnp.concatenate([_dot(onehot, emb_ref[...]), pos], axis=1)

    A0 = _dot(n0, w0s_ref[...]) + b0_ref[...]
    B0 = _dot(n0, w0r_ref[...])
    msg0 = _edge_layer(nuc, A0, B0, w0e_ref[...], v0_ref[...], c0_ref[...], consts)
    h0 = _silu(_dot(n0, u0a_ref[...]) + _dot(msg0, u0b_ref[...]) + ub0_ref[...])
    n1 = _dot(h0, u0c_ref[...]) + uc0_ref[...]

    A1 = _dot(n1, w1s_ref[...]) + b1_ref[...]
    B1 = _dot(n1, w1r_ref[...])
    msg1 = _edge_layer(nuc, A1, B1, w1e_ref[...], v1_ref[...], c1_ref[...], consts)
    h1 = _silu(_dot(n1, u1a_ref[...]) + _dot(msg1, u1b_ref[...]) + ub1_ref[...])
    n2 = n1 + _dot(h1, u1c_ref[...]) + uc1_ref[...]

    ncat = jnp.concatenate([n0, n1, n2], axis=1)
    hn = _silu(_dot(ncat, n0_ref[...]) + nb0_ref[...])
    node = _dot(hn, n1_ref[...]) + _dot(onehot, nemb_ref[...])
    node_out_ref[...] = node

    mean = jnp.mean(ncat, axis=0, keepdims=True)
    hg = _silu(_dot(mean, g0_ref[...]) + gb0_ref[...])
    g = _dot(hg, g1_ref[...]) + gb1_ref[...]
    glob_out_ref[...] = jnp.broadcast_to(g, (8, 128))


def _row(v):
    return v.reshape(1, -1)


@jax.jit
def kernel(nuclei, params):
    nuclei = nuclei.reshape(-1, 3).astype(jnp.float32)
    p = params
    emb = p["emb"]
    d0 = emb.shape[1] + _N_RAD * _N_SPH
    d1 = p["upd0"][1][0].shape[1]

    W0, b0 = p["msg0"][0]
    V0, c0 = p["msg0"][1]
    W1, b1 = p["msg1"][0]
    V1, c1 = p["msg1"][1]
    U0, ub0 = p["upd0"][0]
    U0c, uc0 = p["upd0"][1]
    U1, ub1 = p["upd1"][0]
    U1c, uc1 = p["upd1"][1]
    N0, nb0 = p["node0"][0]
    N1, _ = p["node0"][1]
    G0, gb0 = p["glob0"][0]
    G1, gb1 = p["glob0"][1]

    args = (
        nuclei, emb, p["node_emb"],
        jnp.asarray(_ONEHOT), jnp.asarray(_R), jnp.asarray(_MASK),
        jnp.asarray(_E6), jnp.asarray(_E7),
        W0[:d0], W0[d0:2 * d0], W0[2 * d0:], _row(b0), V0, _row(c0),
        U0[:d0], U0[d0:], _row(ub0), U0c, _row(uc0),
        W1[:d1], W1[d1:2 * d1], W1[2 * d1:], _row(b1), V1, _row(c1),
        U1[:d1], U1[d1:], _row(ub1), U1c, _row(uc1),
        N0, _row(nb0), N1,
        G0, _row(gb0), G1, _row(gb1),
    )
    node_out, gpad = pl.pallas_call(
        _fwd_kernel,
        out_shape=(
            jax.ShapeDtypeStruct((_N, 3), jnp.float32),
            jax.ShapeDtypeStruct((8, 128), jnp.float32),
        ),
    )(*args)
    return node_out, gpad[0, 0:1]


# speedup vs baseline: 2.9999x; 1.5998x over previous
import functools

import jax
import jax.numpy as jnp
import numpy as np
from jax.experimental import pallas as pl

_N = 128
_RBF_DIM = 32
_RBF_CUT = 10.0
_POS_CUT = 5.0
_N_RAD = 6
_N_SPH = 7
_CHARGES = np.tile(np.array([1, 1, 6, 8], dtype=np.int64), 32).astype(np.int32)
_S = 8
_NB = _N // _S
_SP = _S * _N

_ONEHOT = np.zeros((_N, 9), dtype=np.float32)
_ONEHOT[np.arange(_N), _CHARGES] = 1.0
_E6 = np.kron(np.eye(_N_RAD, dtype=np.float32), np.ones((1, _N_SPH), np.float32))
_E7 = np.kron(np.ones((1, _N_RAD), np.float32), np.eye(_N_SPH, dtype=np.float32))
_R = np.kron(np.eye(_S, dtype=np.float32), np.ones((_N, 1), np.float32))
_MASK = np.ones((_NB * _SP, 1), dtype=np.float32)
for _b in range(_NB):
    for _s in range(_S):
        _MASK[_b * _SP + _s * _N + (_b * _S + _s), 0] = 0.0


_DN = (((1,), (0,)), ((), ()))


def _split(x):
    hi = x.astype(jnp.bfloat16)
    lo = (x - hi.astype(jnp.float32)).astype(jnp.bfloat16)
    return hi, lo


def _bdot(a, b, dn=_DN):
    return jax.lax.dot_general(a, b, dn, preferred_element_type=jnp.float32)


def _dot(a, b):
    ah, al = _split(a)
    bh, bl = _split(b)
    return _bdot(ah, bh) + (_bdot(ah, bl) + _bdot(al, bh))


def _dot_exact_lhs(a, b, dn=_DN):
    ab = a.astype(jnp.bfloat16)
    bh, bl = _split(b)
    return _bdot(ab, bh, dn) + _bdot(ab, bl, dn)


def _silu(x):
    return x * jax.nn.sigmoid(x)


def _edge_layer(nuc, A, B, We, V, c, consts):
    R, mask, karr = consts
    B_t = jnp.concatenate([B] * _S, axis=0)
    nr_t = jnp.concatenate([nuc] * _S, axis=0)
    out_blocks = []
    for b in range(_NB):
        Ablk = A[b * _S:(b + 1) * _S, :]
        nblk = nuc[b * _S:(b + 1) * _S, :]
        a_pairs = _dot_exact_lhs(R, Ablk)
        s_pairs = _dot_exact_lhs(R, nblk)
        d = s_pairs - nr_t
        r = jnp.sqrt(jnp.sum(d * d, axis=1, keepdims=True))
        e = (np.float32(np.sqrt(2.0 / _RBF_CUT))
             * jnp.sin(karr * (np.float32(np.pi / _RBF_CUT)) * r)
             / (r + 1e-8))
        pre = a_pairs + B_t + _dot(e, We)
        m = _dot(_silu(pre), V) + c
        m = m * mask[b * _SP:(b + 1) * _SP, :]
        s = _dot_exact_lhs(R, m, (((0,), (0,)), ((), ())))
        out_blocks.append(s)
    msg = jnp.concatenate(out_blocks, axis=0) * np.float32(1.0 / (_N - 1))
    return msg


def _fwd_kernel(nuc_ref, emb_ref, nemb_ref,
                onehot_ref, r_ref, mask_ref, e6_ref, e7_ref,
                w0s_ref, w0r_ref, w0e_ref, b0_ref, v0_ref, c0_ref,
                u0a_ref, u0b_ref, ub0_ref, u0c_ref, uc0_ref,
                w1s_ref, w1r_ref, w1e_ref, b1_ref, v1_ref, c1_ref,
                u1a_ref, u1b_ref, ub1_ref, u1c_ref, uc1_ref,
                n0_ref, nb0_ref, n1_ref,
                g0_ref, gb0_ref, g1_ref, gb1_ref,
                node_out_ref, glob_out_ref):
    nuc = nuc_ref[...]
    onehot = onehot_ref[...]
    karr = (jax.lax.broadcasted_iota(jnp.int32, (1, _RBF_DIM), 1)
            .astype(jnp.float32) + 1.0)
    consts = (r_ref[...], mask_ref[...], karr)

    center = nuc - jnp.mean(nuc, axis=0, keepdims=True)
    r2 = jnp.sum(center * center, axis=1, keepdims=True)
    r = jnp.sqrt(r2)
    inv = 1.0 / (r + 1e-8)
    u = center * inv
    nrad = (jax.lax.broadcasted_iota(jnp.int32, (1, _N_RAD), 1)
            .astype(jnp.float32) + 1.0)
    rad = (np.float32(np.sqrt(2.0 / _POS_CUT))
           * jnp.sin(nrad * np.float32(np.pi / _POS_CUT) * r) * inv)
    ux, uy, uz = u[:, 0:1], u[:, 1:2], u[:, 2:3]
    sph = jnp.concatenate(
        [jnp.ones_like(ux), ux, uy, uz, ux * uy, uy * uz, ux * uz], axis=1)
    pos = _dot(rad, e6_ref[...]) * _dot(sph, e7_ref[...])
    n0 = jnp.concatenate([_dot(onehot, emb_ref[...]), pos], axis=1)

    A0 = _dot(n0, w0s_ref[...]) + b0_ref[...]
    B0 = _dot(n0, w0r_ref[...])
    msg0 = _edge_layer(nuc, A0, B0, w0e_ref[...], v0_ref[...], c0_ref[...], consts)
    h0 = _silu(_dot(n0, u0a_ref[...]) + _dot(msg0, u0b_ref[...]) + ub0_ref[...])
    n1 = _dot(h0, u0c_ref[...]) + uc0_ref[...]

    A1 = _dot(n1, w1s_ref[...]) + b1_ref[...]
    B1 = _dot(n1, w1r_ref[...])
    msg1 = _edge_layer(nuc, A1, B1, w1e_ref[...], v1_ref[...], c1_ref[...], consts)
    h1 = _silu(_dot(n1, u1a_ref[...]) + _dot(msg1, u1b_ref[...]) + ub1_ref[...])
    n2 = n1 + _dot(h1, u1c_ref[...]) + uc1_ref[...]

    ncat = jnp.concatenate([n0, n1, n2], axis=1)
    hn = _silu(_dot(ncat, n0_ref[...]) + nb0_ref[...])
    node = _dot(hn, n1_ref[...]) + _dot(onehot, nemb_ref[...])
    node_out_ref[...] = node

    mean = jnp.mean(ncat, axis=0, keepdims=True)
    hg = _silu(_dot(mean, g0_ref[...]) + gb0_ref[...])
    g = _dot(hg, g1_ref[...]) + gb1_ref[...]
    glob_out_ref[...] = jnp.broadcast_to(g, (8, 128))


def _row(v):
    return v.reshape(1, -1)


@jax.jit
def kernel(nuclei, params):
    nuclei = nuclei.reshape(-1, 3).astype(jnp.float32)
    p = params
    emb = p["emb"]
    d0 = emb.shape[1] + _N_RAD * _N_SPH
    d1 = p["upd0"][1][0].shape[1]

    W0, b0 = p["msg0"][0]
    V0, c0 = p["msg0"][1]
    W1, b1 = p["msg1"][0]
    V1, c1 = p["msg1"][1]
    U0, ub0 = p["upd0"][0]
    U0c, uc0 = p["upd0"][1]
    U1, ub1 = p["upd1"][0]
    U1c, uc1 = p["upd1"][1]
    N0, nb0 = p["node0"][0]
    N1, _ = p["node0"][1]
    G0, gb0 = p["glob0"][0]
    G1, gb1 = p["glob0"][1]

    args = (
        nuclei, emb, p["node_emb"],
        jnp.asarray(_ONEHOT), jnp.asarray(_R), jnp.asarray(_MASK),
        jnp.asarray(_E6), jnp.asarray(_E7),
        W0[:d0], W0[d0:2 * d0], W0[2 * d0:], _row(b0), V0, _row(c0),
        U0[:d0], U0[d0:], _row(ub0), U0c, _row(uc0),
        W1[:d1], W1[d1:2 * d1], W1[2 * d1:], _row(b1), V1, _row(c1),
        U1[:d1], U1[d1:], _row(ub1), U1c, _row(uc1),
        N0, _row(nb0), N1,
        G0, _row(gb0), G1, _row(gb1),
    )
    node_out, gpad = pl.pallas_call(
        _fwd_kernel,
        out_shape=(
            jax.ShapeDtypeStruct((_N, 3), jnp.float32),
            jax.ShapeDtypeStruct((8, 128), jnp.float32),
        ),
    )(*args)
    return node_out, gpad[0, 0:1]


# speedup vs baseline: 3.1307x; 1.0436x over previous
import functools

import jax
import jax.numpy as jnp
import numpy as np
from jax.experimental import pallas as pl

_N = 128
_RBF_DIM = 32
_RBF_CUT = 10.0
_POS_CUT = 5.0
_N_RAD = 6
_N_SPH = 7
_CHARGES = np.tile(np.array([1, 1, 6, 8], dtype=np.int64), 32).astype(np.int32)
_S = 16
_NB = _N // _S
_SP = _S * _N

_ONEHOT = np.zeros((_N, 9), dtype=np.float32)
_ONEHOT[np.arange(_N), _CHARGES] = 1.0
_E6 = np.kron(np.eye(_N_RAD, dtype=np.float32), np.ones((1, _N_SPH), np.float32))
_E7 = np.kron(np.ones((1, _N_RAD), np.float32), np.eye(_N_SPH, dtype=np.float32))
_R = np.kron(np.eye(_S, dtype=np.float32), np.ones((_N, 1), np.float32))
_MASK = np.ones((_NB * _SP, 1), dtype=np.float32)
for _b in range(_NB):
    for _s in range(_S):
        _MASK[_b * _SP + _s * _N + (_b * _S + _s), 0] = 0.0


_DN = (((1,), (0,)), ((), ()))


def _split(x):
    hi = x.astype(jnp.bfloat16)
    lo = (x - hi.astype(jnp.float32)).astype(jnp.bfloat16)
    return hi, lo


def _bdot(a, b, dn=_DN):
    return jax.lax.dot_general(a, b, dn, preferred_element_type=jnp.float32)


def _dot(a, b):
    ah, al = _split(a)
    bh, bl = _split(b)
    return _bdot(ah, bh) + (_bdot(ah, bl) + _bdot(al, bh))


def _dot_exact_lhs(a, b, dn=_DN):
    ab = a.astype(jnp.bfloat16)
    bh, bl = _split(b)
    return _bdot(ab, bh, dn) + _bdot(ab, bl, dn)


def _silu(x):
    return x * jax.nn.sigmoid(x)


def _edge_layer(nuc, A, B, We, V, c, consts):
    R, mask, karr = consts
    B_t = jnp.concatenate([B] * _S, axis=0)
    nr_t = jnp.concatenate([nuc] * _S, axis=0)
    out_blocks = []
    for b in range(_NB):
        Ablk = A[b * _S:(b + 1) * _S, :]
        nblk = nuc[b * _S:(b + 1) * _S, :]
        a_pairs = _dot_exact_lhs(R, Ablk)
        s_pairs = _dot_exact_lhs(R, nblk)
        d = s_pairs - nr_t
        r = jnp.sqrt(jnp.sum(d * d, axis=1, keepdims=True))
        e = (np.float32(np.sqrt(2.0 / _RBF_CUT))
             * jnp.sin(karr * (np.float32(np.pi / _RBF_CUT)) * r)
             / (r + 1e-8))
        pre = a_pairs + B_t + _dot(e, We)
        m = _dot(_silu(pre), V) + c
        m = m * mask[b * _SP:(b + 1) * _SP, :]
        s = _dot_exact_lhs(R, m, (((0,), (0,)), ((), ())))
        out_blocks.append(s)
    msg = jnp.concatenate(out_blocks, axis=0) * np.float32(1.0 / (_N - 1))
    return msg


def _fwd_kernel(nuc_ref, emb_ref, nemb_ref,
                onehot_ref, r_ref, mask_ref, e6_ref, e7_ref,
                w0s_ref, w0r_ref, w0e_ref, b0_ref, v0_ref, c0_ref,
                u0a_ref, u0b_ref, ub0_ref, u0c_ref, uc0_ref,
                w1s_ref, w1r_ref, w1e_ref, b1_ref, v1_ref, c1_ref,
                u1a_ref, u1b_ref, ub1_ref, u1c_ref, uc1_ref,
                n0_ref, nb0_ref, n1_ref,
                g0_ref, gb0_ref, g1_ref, gb1_ref,
                node_out_ref, glob_out_ref):
    nuc = nuc_ref[...]
    onehot = onehot_ref[...]
    karr = (jax.lax.broadcasted_iota(jnp.int32, (1, _RBF_DIM), 1)
            .astype(jnp.float32) + 1.0)
    consts = (r_ref[...], mask_ref[...], karr)

    center = nuc - jnp.mean(nuc, axis=0, keepdims=True)
    r2 = jnp.sum(center * center, axis=1, keepdims=True)
    r = jnp.sqrt(r2)
    inv = 1.0 / (r + 1e-8)
    u = center * inv
    nrad = (jax.lax.broadcasted_iota(jnp.int32, (1, _N_RAD), 1)
            .astype(jnp.float32) + 1.0)
    rad = (np.float32(np.sqrt(2.0 / _POS_CUT))
           * jnp.sin(nrad * np.float32(np.pi / _POS_CUT) * r) * inv)
    ux, uy, uz = u[:, 0:1], u[:, 1:2], u[:, 2:3]
    sph = jnp.concatenate(
        [jnp.ones_like(ux), ux, uy, uz, ux * uy, uy * uz, ux * uz], axis=1)
    pos = _dot(rad, e6_ref[...]) * _dot(sph, e7_ref[...])
    n0 = jnp.concatenate([_dot(onehot, emb_ref[...]), pos], axis=1)

    A0 = _dot(n0, w0s_ref[...]) + b0_ref[...]
    B0 = _dot(n0, w0r_ref[...])
    msg0 = _edge_layer(nuc, A0, B0, w0e_ref[...], v0_ref[...], c0_ref[...], consts)
    h0 = _silu(_dot(n0, u0a_ref[...]) + _dot(msg0, u0b_ref[...]) + ub0_ref[...])
    n1 = _dot(h0, u0c_ref[...]) + uc0_ref[...]

    A1 = _dot(n1, w1s_ref[...]) + b1_ref[...]
    B1 = _dot(n1, w1r_ref[...])
    msg1 = _edge_layer(nuc, A1, B1, w1e_ref[...], v1_ref[...], c1_ref[...], consts)
    h1 = _silu(_dot(n1, u1a_ref[...]) + _dot(msg1, u1b_ref[...]) + ub1_ref[...])
    n2 = n1 + _dot(h1, u1c_ref[...]) + uc1_ref[...]

    ncat = jnp.concatenate([n0, n1, n2], axis=1)
    hn = _silu(_dot(ncat, n0_ref[...]) + nb0_ref[...])
    node = _dot(hn, n1_ref[...]) + _dot(onehot, nemb_ref[...])
    node_out_ref[...] = node

    mean = jnp.mean(ncat, axis=0, keepdims=True)
    hg = _silu(_dot(mean, g0_ref[...]) + gb0_ref[...])
    g = _dot(hg, g1_ref[...]) + gb1_ref[...]
    glob_out_ref[...] = jnp.broadcast_to(g, (8, 128))


def _row(v):
    return v.reshape(1, -1)


@jax.jit
def kernel(nuclei, params):
    nuclei = nuclei.reshape(-1, 3).astype(jnp.float32)
    p = params
    emb = p["emb"]
    d0 = emb.shape[1] + _N_RAD * _N_SPH
    d1 = p["upd0"][1][0].shape[1]

    W0, b0 = p["msg0"][0]
    V0, c0 = p["msg0"][1]
    W1, b1 = p["msg1"][0]
    V1, c1 = p["msg1"][1]
    U0, ub0 = p["upd0"][0]
    U0c, uc0 = p["upd0"][1]
    U1, ub1 = p["upd1"][0]
    U1c, uc1 = p["upd1"][1]
    N0, nb0 = p["node0"][0]
    N1, _ = p["node0"][1]
    G0, gb0 = p["glob0"][0]
    G1, gb1 = p["glob0"][1]

    args = (
        nuclei, emb, p["node_emb"],
        jnp.asarray(_ONEHOT), jnp.asarray(_R), jnp.asarray(_MASK),
        jnp.asarray(_E6), jnp.asarray(_E7),
        W0[:d0], W0[d0:2 * d0], W0[2 * d0:], _row(b0), V0, _row(c0),
        U0[:d0], U0[d0:], _row(ub0), U0c, _row(uc0),
        W1[:d1], W1[d1:2 * d1], W1[2 * d1:], _row(b1), V1, _row(c1),
        U1[:d1], U1[d1:], _row(ub1), U1c, _row(uc1),
        N0, _row(nb0), N1,
        G0, _row(gb0), G1, _row(gb1),
    )
    node_out, gpad = pl.pallas_call(
        _fwd_kernel,
        out_shape=(
            jax.ShapeDtypeStruct((_N, 3), jnp.float32),
            jax.ShapeDtypeStruct((8, 128), jnp.float32),
        ),
    )(*args)
    return node_out, gpad[0, 0:1]


# speedup vs baseline: 4.0790x; 1.3029x over previous
import functools

import jax
import jax.numpy as jnp
import numpy as np
from jax.experimental import pallas as pl

_N = 128
_RBF_DIM = 32
_RBF_CUT = 10.0
_POS_CUT = 5.0
_N_RAD = 6
_N_SPH = 7
_CHARGES = np.tile(np.array([1, 1, 6, 8], dtype=np.int64), 32).astype(np.int32)
_S = 16
_NB = _N // _S
_SP = _S * _N

_ONEHOT = np.zeros((_N, 9), dtype=np.float32)
_ONEHOT[np.arange(_N), _CHARGES] = 1.0
_E6 = np.kron(np.eye(_N_RAD, dtype=np.float32), np.ones((1, _N_SPH), np.float32))
_E7 = np.kron(np.ones((1, _N_RAD), np.float32), np.eye(_N_SPH, dtype=np.float32))
_R = np.kron(np.eye(_S, dtype=np.float32), np.ones((_N, 1), np.float32))
_MASK = np.ones((_NB * _SP, 1), dtype=np.float32)
for _b in range(_NB):
    for _s in range(_S):
        _MASK[_b * _SP + _s * _N + (_b * _S + _s), 0] = 0.0


_DN = (((1,), (0,)), ((), ()))


def _split(x):
    hi = x.astype(jnp.bfloat16)
    lo = (x - hi.astype(jnp.float32)).astype(jnp.bfloat16)
    return hi, lo


def _bdot(a, b, dn=_DN):
    return jax.lax.dot_general(a, b, dn, preferred_element_type=jnp.float32)


def _dot(a, b):
    ah, al = _split(a)
    bh, bl = _split(b)
    return _bdot(ah, bh) + (_bdot(ah, bl) + _bdot(al, bh))


def _dot_exact_lhs(a, b, dn=_DN):
    ab = a.astype(jnp.bfloat16)
    bh, bl = _split(b)
    return _bdot(ab, bh, dn) + _bdot(ab, bl, dn)


def _silu(x):
    return x * jax.nn.sigmoid(x)


_DNT = (((0,), (0,)), ((), ()))


def _pair_geometry(nuc, Rb, karr):
    nr_t = jnp.concatenate([nuc] * _S, axis=0)
    ones_col = jnp.ones((_SP, 1), jnp.bfloat16)
    L1s, L2s = [], []
    for b in range(_NB):
        nblk = nuc[b * _S:(b + 1) * _S, :]
        nh, nl = _split(nblk)
        s_pairs = _bdot(Rb, nh) + _bdot(Rb, nl)
        d = s_pairs - nr_t
        r = jnp.sqrt(jnp.sum(d * d, axis=1, keepdims=True))
        e = (np.float32(np.sqrt(2.0 / _RBF_CUT))
             * jnp.sin(karr * (np.float32(np.pi / _RBF_CUT)) * r)
             / (r + 1e-8))
        eh, el = _split(e)
        L1s.append(jnp.concatenate([Rb, eh, el, ones_col], axis=1))
        L2s.append(jnp.concatenate([Rb, eh, ones_col], axis=1))
    return L1s, L2s


def _edge_layer(L1s, L2s, Rb, mask, A, B, We, b0, V, c):
    Ah, Al = _split(A)
    Weh, Wel = _split(We)
    bh, bl = _split(b0)
    Vh, Vl = _split(V)
    B_t = jnp.concatenate([B] * _S, axis=0)
    out_blocks = []
    for b in range(_NB):
        rhs1 = jnp.concatenate([Ah[b * _S:(b + 1) * _S], Weh, Weh, bh], axis=0)
        rhs2 = jnp.concatenate([Al[b * _S:(b + 1) * _S], Wel, bl], axis=0)
        pre = _bdot(L1s[b], rhs1) + _bdot(L2s[b], rhs2) + B_t
        act = _silu(pre).astype(jnp.bfloat16)
        m = _bdot(act, Vh) + _bdot(act, Vl) + c
        m = m * mask[b * _SP:(b + 1) * _SP, :]
        mh, ml = _split(m)
        out_blocks.append(_bdot(Rb, mh, _DNT) + _bdot(Rb, ml, _DNT))
    msg = jnp.concatenate(out_blocks, axis=0) * np.float32(1.0 / (_N - 1))
    return msg


def _fwd_kernel(nuc_ref, emb_ref, nemb_ref,
                onehot_ref, r_ref, mask_ref, e6_ref, e7_ref,
                w0s_ref, w0r_ref, w0e_ref, b0_ref, v0_ref, c0_ref,
                u0a_ref, u0b_ref, ub0_ref, u0c_ref, uc0_ref,
                w1s_ref, w1r_ref, w1e_ref, b1_ref, v1_ref, c1_ref,
                u1a_ref, u1b_ref, ub1_ref, u1c_ref, uc1_ref,
                n0_ref, nb0_ref, n1_ref,
                g0_ref, gb0_ref, g1_ref, gb1_ref,
                node_out_ref, glob_out_ref):
    nuc = nuc_ref[...]
    onehot = onehot_ref[...]
    karr = (jax.lax.broadcasted_iota(jnp.int32, (1, _RBF_DIM), 1)
            .astype(jnp.float32) + 1.0)
    Rb = r_ref[...].astype(jnp.bfloat16)
    mask = mask_ref[...]
    L1s, L2s = _pair_geometry(nuc, Rb, karr)

    center = nuc - jnp.mean(nuc, axis=0, keepdims=True)
    r2 = jnp.sum(center * center, axis=1, keepdims=True)
    r = jnp.sqrt(r2)
    inv = 1.0 / (r + 1e-8)
    u = center * inv
    nrad = (jax.lax.broadcasted_iota(jnp.int32, (1, _N_RAD), 1)
            .astype(jnp.float32) + 1.0)
    rad = (np.float32(np.sqrt(2.0 / _POS_CUT))
           * jnp.sin(nrad * np.float32(np.pi / _POS_CUT) * r) * inv)
    ux, uy, uz = u[:, 0:1], u[:, 1:2], u[:, 2:3]
    sph = jnp.concatenate(
        [jnp.ones_like(ux), ux, uy, uz, ux * uy, uy * uz, ux * uz], axis=1)
    pos = _dot(rad, e6_ref[...]) * _dot(sph, e7_ref[...])
    n0 = jnp.concatenate([_dot(onehot, emb_ref[...]), pos], axis=1)

    A0 = _dot(n0, w0s_ref[...])
    B0 = _dot(n0, w0r_ref[...])
    msg0 = _edge_layer(L1s, L2s, Rb, mask, A0, B0, w0e_ref[...],
                       b0_ref[...], v0_ref[...], c0_ref[...])
    h0 = _silu(_dot(n0, u0a_ref[...]) + _dot(msg0, u0b_ref[...]) + ub0_ref[...])
    n1 = _dot(h0, u0c_ref[...]) + uc0_ref[...]

    A1 = _dot(n1, w1s_ref[...])
    B1 = _dot(n1, w1r_ref[...])
    msg1 = _edge_layer(L1s, L2s, Rb, mask, A1, B1, w1e_ref[...],
                       b1_ref[...], v1_ref[...], c1_ref[...])
    h1 = _silu(_dot(n1, u1a_ref[...]) + _dot(msg1, u1b_ref[...]) + ub1_ref[...])
    n2 = n1 + _dot(h1, u1c_ref[...]) + uc1_ref[...]

    ncat = jnp.concatenate([n0, n1, n2], axis=1)
    hn = _silu(_dot(ncat, n0_ref[...]) + nb0_ref[...])
    node = _dot(hn, n1_ref[...]) + _dot(onehot, nemb_ref[...])
    node_out_ref[...] = node

    mean = jnp.mean(ncat, axis=0, keepdims=True)
    hg = _silu(_dot(mean, g0_ref[...]) + gb0_ref[...])
    g = _dot(hg, g1_ref[...]) + gb1_ref[...]
    glob_out_ref[...] = jnp.broadcast_to(g, (8, 128))


def _row(v):
    return v.reshape(1, -1)


@jax.jit
def kernel(nuclei, params):
    nuclei = nuclei.reshape(-1, 3).astype(jnp.float32)
    p = params
    emb = p["emb"]
    d0 = emb.shape[1] + _N_RAD * _N_SPH
    d1 = p["upd0"][1][0].shape[1]

    W0, b0 = p["msg0"][0]
    V0, c0 = p["msg0"][1]
    W1, b1 = p["msg1"][0]
    V1, c1 = p["msg1"][1]
    U0, ub0 = p["upd0"][0]
    U0c, uc0 = p["upd0"][1]
    U1, ub1 = p["upd1"][0]
    U1c, uc1 = p["upd1"][1]
    N0, nb0 = p["node0"][0]
    N1, _ = p["node0"][1]
    G0, gb0 = p["glob0"][0]
    G1, gb1 = p["glob0"][1]

    args = (
        nuclei, emb, p["node_emb"],
        jnp.asarray(_ONEHOT), jnp.asarray(_R), jnp.asarray(_MASK),
        jnp.asarray(_E6), jnp.asarray(_E7),
        W0[:d0], W0[d0:2 * d0], W0[2 * d0:], _row(b0), V0, _row(c0),
        U0[:d0], U0[d0:], _row(ub0), U0c, _row(uc0),
        W1[:d1], W1[d1:2 * d1], W1[2 * d1:], _row(b1), V1, _row(c1),
        U1[:d1], U1[d1:], _row(ub1), U1c, _row(uc1),
        N0, _row(nb0), N1,
        G0, _row(gb0), G1, _row(gb1),
    )
    node_out, gpad = pl.pallas_call(
        _fwd_kernel,
        out_shape=(
            jax.ShapeDtypeStruct((_N, 3), jnp.float32),
            jax.ShapeDtypeStruct((8, 128), jnp.float32),
        ),
    )(*args)
    return node_out, gpad[0, 0:1]
